# baseline (speedup 1.0000x reference)
import functools

import jax
import jax.numpy as jnp
from jax import lax
from jax.experimental import pallas as pl
from jax.experimental.pallas import tpu as pltpu
from jax.experimental.pallas import tpu_sc as plsc

USER_NUM = 1000000
ITEM_NUM = 100000
BATCH = 16384
VALUE_RANGE = 8.0
A_RANGE = 4.0
D_CONST = 1.702

NC = 2
NS = 16
L = 16
NW = NC * NS
B_PER_W = BATCH // NW
CHUNK = 128
NCHUNK = B_PER_W // CHUNK
NROW = 5

A_BASE = USER_NUM
B_BASE = USER_NUM + ITEM_NUM
C_BASE = USER_NUM + 2 * ITEM_NUM


def _body(ids_hbm, tab_hbm, out_hbm, ids_v, th_v, a_v, b_v, c_v, res_v,
          sem0, sem1, sem2, sem3, semo):
    wid = lax.axis_index("s") * NC + lax.axis_index("c")
    sems = (sem0, sem1, sem2, sem3)
    dests = (th_v, a_v, b_v, c_v)

    pltpu.async_copy(ids_hbm.at[wid], ids_v, sem0).wait()

    copies = [[] for _ in range(NCHUNK)]
    for j in range(NCHUNK):
        dst = pl.ds(j * CHUNK, CHUNK)
        for t in range(4):
            copies[j].append(pltpu.async_copy(
                tab_hbm.at[ids_v.at[t].at[j]], dests[t].at[dst], sems[j]))

    fair_ne0 = ids_v[4, 0, pl.ds(0, L)] != 0

    out_cps = []
    for j in range(NCHUNK):
        for cp in copies[j]:
            cp.wait()

        @plsc.parallel_loop(j * CHUNK, (j + 1) * CHUNK, step=L, unroll=2)
        def _compute(i):
            sl = pl.ds(i, L)
            sig_t = 1.0 / (1.0 + jnp.exp(-th_v[sl]))
            theta = VALUE_RANGE * (sig_t - 0.5)
            a = A_RANGE / (1.0 + jnp.exp(-a_v[sl]))
            b = VALUE_RANGE / (1.0 + jnp.exp(-b_v[sl])) - 4.0
            ez = jnp.exp(-D_CONST * a * (theta - b))
            u = 1.0 + jnp.exp(-c_v[sl])
            irf = (ez + u) / (u * (1.0 + ez))
            res_v[sl] = jnp.where(fair_ne0, sig_t, irf)

        dst = pl.ds(j * CHUNK, CHUNK)
        out_cps.append(
            pltpu.async_copy(res_v.at[dst], out_hbm.at[wid].at[dst], semo))
    for cp in out_cps:
        cp.wait()


@functools.partial(
    pl.kernel,
    mesh=plsc.VectorSubcoreMesh(core_axis_name="c", subcore_axis_name="s"),
    out_type=jax.ShapeDtypeStruct((NW, B_PER_W), jnp.float32),
    scratch_types=[
        pltpu.VMEM((NROW, NCHUNK, CHUNK), jnp.int32),
        pltpu.VMEM((B_PER_W,), jnp.float32),
        pltpu.VMEM((B_PER_W,), jnp.float32),
        pltpu.VMEM((B_PER_W,), jnp.float32),
        pltpu.VMEM((B_PER_W,), jnp.float32),
        pltpu.VMEM((B_PER_W,), jnp.float32),
        pltpu.SemaphoreType.DMA,
        pltpu.SemaphoreType.DMA,
        pltpu.SemaphoreType.DMA,
        pltpu.SemaphoreType.DMA,
        pltpu.SemaphoreType.DMA,
    ],
)
def _irt_sc_kernel(ids, tab, out, *scratch):
    _body(ids, tab, out, *scratch)


def kernel(user, item, fairness, theta_table, a_table, b_table, c_table):
    shp = (NW, NCHUNK, CHUNK)
    fair_i32 = jnp.asarray(fairness, jnp.int32).reshape(())
    ids = jnp.stack([
        user.reshape(shp),
        (item + A_BASE).reshape(shp),
        (item + B_BASE).reshape(shp),
        (item + C_BASE).reshape(shp),
        jnp.broadcast_to(fair_i32, shp),
    ], axis=1)
    tab = jnp.concatenate([
        theta_table.reshape(USER_NUM),
        a_table.reshape(ITEM_NUM),
        b_table.reshape(ITEM_NUM),
        c_table.reshape(ITEM_NUM),
    ])
    out = _irt_sc_kernel(ids, tab)
    return out.reshape(BATCH)


# speedup vs baseline: 1.2561x; 1.2561x over previous
import functools

import jax
import jax.numpy as jnp
from jax import lax
from jax.experimental import pallas as pl
from jax.experimental.pallas import tpu as pltpu
from jax.experimental.pallas import tpu_sc as plsc

USER_NUM = 1000000
ITEM_NUM = 100000
BATCH = 16384
VALUE_RANGE = 8.0
A_RANGE = 4.0
D_CONST = 1.702

NC = 2
NS = 16
L = 16
NW = NC * NS
B_PER_W = BATCH // NW
CHUNK = 128
NCHUNK = B_PER_W // CHUNK
NROW = 5

A_BASE = 0
B_BASE = ITEM_NUM
C_BASE = 2 * ITEM_NUM


def _body(ids_hbm, theta_hbm, abc_hbm, out_hbm, ids_v, th_v, a_v, b_v, c_v,
          res_v, sem0, sem1, sem2, sem3, semo):
    wid = lax.axis_index("s") * NC + lax.axis_index("c")
    sems = (sem0, sem1, sem2, sem3)
    dests = (th_v, a_v, b_v, c_v)
    tabs = (theta_hbm, abc_hbm, abc_hbm, abc_hbm)

    pltpu.async_copy(ids_hbm.at[wid], ids_v, sem0).wait()

    copies = [[] for _ in range(NCHUNK)]
    for j in range(NCHUNK):
        dst = pl.ds(j * CHUNK, CHUNK)
        for t in range(4):
            copies[j].append(pltpu.async_copy(
                tabs[t].at[ids_v.at[t].at[j]], dests[t].at[dst], sems[j]))

    fair_ne0 = ids_v[4, 0, pl.ds(0, L)] != 0

    out_cps = []
    for j in range(NCHUNK):
        for cp in copies[j]:
            cp.wait()

        @plsc.parallel_loop(j * CHUNK, (j + 1) * CHUNK, step=L, unroll=2)
        def _compute(i):
            sl = pl.ds(i, L)
            sig_t = 1.0 / (1.0 + jnp.exp(-th_v[sl]))
            theta = VALUE_RANGE * (sig_t - 0.5)
            a = A_RANGE / (1.0 + jnp.exp(-a_v[sl]))
            b = VALUE_RANGE / (1.0 + jnp.exp(-b_v[sl])) - 4.0
            ez = jnp.exp(-D_CONST * a * (theta - b))
            u = 1.0 + jnp.exp(-c_v[sl])
            irf = (ez + u) / (u * (1.0 + ez))
            res_v[sl] = jnp.where(fair_ne0, sig_t, irf)

        dst = pl.ds(j * CHUNK, CHUNK)
        out_cps.append(
            pltpu.async_copy(res_v.at[dst], out_hbm.at[wid].at[dst], semo))
    for cp in out_cps:
        cp.wait()


@functools.partial(
    pl.kernel,
    mesh=plsc.VectorSubcoreMesh(core_axis_name="c", subcore_axis_name="s"),
    out_type=jax.ShapeDtypeStruct((NW, B_PER_W), jnp.float32),
    scratch_types=[
        pltpu.VMEM((NROW, NCHUNK, CHUNK), jnp.int32),
        pltpu.VMEM((B_PER_W,), jnp.float32),
        pltpu.VMEM((B_PER_W,), jnp.float32),
        pltpu.VMEM((B_PER_W,), jnp.float32),
        pltpu.VMEM((B_PER_W,), jnp.float32),
        pltpu.VMEM((B_PER_W,), jnp.float32),
        pltpu.SemaphoreType.DMA,
        pltpu.SemaphoreType.DMA,
        pltpu.SemaphoreType.DMA,
        pltpu.SemaphoreType.DMA,
        pltpu.SemaphoreType.DMA,
    ],
)
def _irt_sc_kernel(ids, theta_tab, abc_tab, out, *scratch):
    _body(ids, theta_tab, abc_tab, out, *scratch)


def kernel(user, item, fairness, theta_table, a_table, b_table, c_table):
    shp = (NW, NCHUNK, CHUNK)
    fair_i32 = jnp.asarray(fairness, jnp.int32).reshape(())
    ids = jnp.stack([
        user.reshape(shp),
        (item + A_BASE).reshape(shp),
        (item + B_BASE).reshape(shp),
        (item + C_BASE).reshape(shp),
        jnp.broadcast_to(fair_i32, shp),
    ], axis=1)
    abc = jnp.concatenate([
        a_table.reshape(ITEM_NUM),
        b_table.reshape(ITEM_NUM),
        c_table.reshape(ITEM_NUM),
    ])
    out = _irt_sc_kernel(ids, theta_table.reshape(USER_NUM), abc)
    return out.reshape(BATCH)


# speedup vs baseline: 1.4224x; 1.1324x over previous
import functools

import jax
import jax.numpy as jnp
from jax import lax
from jax.experimental import pallas as pl
from jax.experimental.pallas import tpu as pltpu
from jax.experimental.pallas import tpu_sc as plsc

USER_NUM = 1000000
ITEM_NUM = 100000
BATCH = 16384
VALUE_RANGE = 8.0
A_RANGE = 4.0
D_CONST = 1.702

NC = 2
NS = 16
L = 16
NW = NC * NS
B_PER_W = BATCH // NW
CHUNK = 128
NCHUNK = B_PER_W // CHUNK


def _abc_body(item_hbm, a_hbm, b_hbm, c_hbm, out_hbm,
              iidx_v, abc_v, sem0, sem1, sem2, sem3, semo):
    wid = lax.axis_index("s") * NC + lax.axis_index("c")
    sems = (sem0, sem1, sem2, sem3)
    tabs = (a_hbm, b_hbm, c_hbm)

    pltpu.async_copy(item_hbm.at[wid], iidx_v, sem0).wait()

    copies = []
    for j in range(NCHUNK):
        for t in range(3):
            dst = pl.ds(t * B_PER_W + j * CHUNK, CHUNK)
            copies.append(
                pltpu.async_copy(tabs[t].at[iidx_v.at[j]], abc_v.at[dst], sems[j]))
    for cp in copies:
        cp.wait()

    pltpu.async_copy(
        abc_v, out_hbm.at[pl.ds(wid * 3 * B_PER_W, 3 * B_PER_W)], semo).wait()


@functools.partial(
    pl.kernel,
    mesh=plsc.VectorSubcoreMesh(core_axis_name="c", subcore_axis_name="s"),
    out_type=jax.ShapeDtypeStruct((3 * BATCH,), jnp.float32),
    scratch_types=[
        pltpu.VMEM((NCHUNK, CHUNK), jnp.int32),
        pltpu.VMEM((3 * B_PER_W,), jnp.float32),
        pltpu.SemaphoreType.DMA,
        pltpu.SemaphoreType.DMA,
        pltpu.SemaphoreType.DMA,
        pltpu.SemaphoreType.DMA,
        pltpu.SemaphoreType.DMA,
    ],
)
def _abc_gather_kernel(item, a_tab, b_tab, c_tab, out, *scratch):
    _abc_body(item, a_tab, b_tab, c_tab, out, *scratch)


def _irt_body(user_hbm, abc_hbm, fair_hbm, theta_hbm, out_hbm,
              uidx_v, abc_v, th_v, res_v, fair_v,
              sem0, sem1, sem2, sem3, semf, semo):
    wid = lax.axis_index("s") * NC + lax.axis_index("c")
    sems = (sem0, sem1, sem2, sem3)

    ucp = pltpu.async_copy(user_hbm.at[wid], uidx_v, sem0)
    acp = pltpu.async_copy(
        abc_hbm.at[pl.ds(wid * 3 * B_PER_W, 3 * B_PER_W)], abc_v, semf)

    ucp.wait()
    copies = []
    for j in range(NCHUNK):
        dst = pl.ds(j * CHUNK, CHUNK)
        copies.append(
            pltpu.async_copy(theta_hbm.at[uidx_v.at[j]], th_v.at[dst], sems[j]))

    fcp = pltpu.async_copy(fair_hbm, fair_v, semf)
    acp.wait()
    fcp.wait()
    fair_ne0 = fair_v[...] != 0

    out_cps = []
    for j in range(NCHUNK):
        copies[j].wait()

        @plsc.parallel_loop(j * CHUNK, (j + 1) * CHUNK, step=L, unroll=2)
        def _compute(i):
            sl = pl.ds(i, L)
            sig_t = 1.0 / (1.0 + jnp.exp(-th_v[sl]))
            theta = VALUE_RANGE * (sig_t - 0.5)
            a = A_RANGE / (1.0 + jnp.exp(-abc_v[pl.ds(i, L)]))
            b = VALUE_RANGE / (1.0 + jnp.exp(-abc_v[pl.ds(B_PER_W + i, L)])) - 4.0
            ez = jnp.exp(-D_CONST * a * (theta - b))
            u = 1.0 + jnp.exp(-abc_v[pl.ds(2 * B_PER_W + i, L)])
            irf = (ez + u) / (u * (1.0 + ez))
            res_v[sl] = jnp.where(fair_ne0, sig_t, irf)

        dst = pl.ds(j * CHUNK, CHUNK)
        out_cps.append(
            pltpu.async_copy(res_v.at[dst], out_hbm.at[wid].at[dst], semo))
    for cp in out_cps:
        cp.wait()


@functools.partial(
    pl.kernel,
    mesh=plsc.VectorSubcoreMesh(core_axis_name="c", subcore_axis_name="s"),
    out_type=jax.ShapeDtypeStruct((NW, B_PER_W), jnp.float32),
    scratch_types=[
        pltpu.VMEM((NCHUNK, CHUNK), jnp.int32),
        pltpu.VMEM((3 * B_PER_W,), jnp.float32),
        pltpu.VMEM((B_PER_W,), jnp.float32),
        pltpu.VMEM((B_PER_W,), jnp.float32),
        pltpu.VMEM((L,), jnp.int32),
        pltpu.SemaphoreType.DMA,
        pltpu.SemaphoreType.DMA,
        pltpu.SemaphoreType.DMA,
        pltpu.SemaphoreType.DMA,
        pltpu.SemaphoreType.DMA,
        pltpu.SemaphoreType.DMA,
    ],
)
def _irt_kernel(user, abc, fair, theta_tab, out, *scratch):
    _irt_body(user, abc, fair, theta_tab, out, *scratch)


def kernel(user, item, fairness, theta_table, a_table, b_table, c_table):
    user3 = user.reshape(NW, NCHUNK, CHUNK)
    item3 = item.reshape(NW, NCHUNK, CHUNK)
    fair_vec = jnp.broadcast_to(
        jnp.asarray(fairness, jnp.int32).reshape(()), (L,))
    abc = _abc_gather_kernel(
        item3,
        a_table.reshape(ITEM_NUM),
        b_table.reshape(ITEM_NUM),
        c_table.reshape(ITEM_NUM),
    )
    out = _irt_kernel(user3, abc, fair_vec, theta_table.reshape(USER_NUM))
    return out.reshape(BATCH)
